# baseline (speedup 1.0000x reference)
import jax
import jax.numpy as jnp
from jax import lax
from jax.experimental import pallas as pl
from jax.experimental.pallas import tpu as pltpu
from jax.experimental.pallas import tpu_sc as plsc

D = 128
EPS = 1e-12
NC = 2
NS = 16
NW = NC * NS
CHUNK = 128
NVR = D // 16


def _rsqrt_newton(x):
    i = plsc.bitcast(x, jnp.int32)
    i = jnp.int32(0x5F3759DF) - lax.shift_right_arithmetic(i, 1)
    y = plsc.bitcast(i, jnp.float32)
    half = jnp.float32(0.5) * x
    for _ in range(3):
        y = y * (jnp.float32(1.5) - half * y * y)
    return y


def _fused_body(idx_hbm, tid_hbm, table_hbm, ttab_hbm, pos_hbm, gam_hbm,
                bet_hbm, out_hbm, idx_v, tid_v, wbuf0, wbuf1, pbuf0, pbuf1,
                const_v, sem0, sem1, psem0, psem1):
    wid = lax.axis_index("s") * NC + lax.axis_index("c")
    n_chunks = idx_hbm.shape[0] // NW
    rows_per_w = n_chunks * CHUNK
    base_row = wid * rows_per_w
    seq_len = pos_hbm.shape[0]
    pos0 = (wid * rows_per_w) % seq_len

    pltpu.sync_copy(idx_hbm.at[pl.ds(wid * n_chunks, n_chunks)], idx_v)
    pltpu.sync_copy(tid_hbm.at[pl.ds(wid * n_chunks, n_chunks)], tid_v)
    pltpu.sync_copy(gam_hbm, const_v.at[0])
    pltpu.sync_copy(bet_hbm, const_v.at[1])
    pltpu.sync_copy(ttab_hbm.at[0], const_v.at[2])
    pltpu.sync_copy(ttab_hbm.at[1], const_v.at[3])

    wbufs = (wbuf0, wbuf1)
    pbufs = (pbuf0, pbuf1)
    wsems = (sem0, sem1)
    psems = (psem0, psem1)

    scale = jnp.float32(float(D) ** 0.5)
    inv_d = jnp.float32(1.0 / D)

    def start(j, b):
        w = pltpu.async_copy(table_hbm.at[idx_v.at[j]], wbufs[b], wsems[b])
        p = pltpu.async_copy(pos_hbm.at[pl.ds(pos0 + j * CHUNK, CHUNK)],
                             pbufs[b], psems[b])
        return (w, p)

    cp = start(0, 0)

    for j in range(n_chunks):
        b = j & 1
        nxt = start(j + 1, 1 - b) if j + 1 < n_chunks else None
        cp[0].wait()
        cp[1].wait()
        wb = wbufs[b]
        pb = pbufs[b]
        tid_row = tid_v.at[j]

        def token(i, carry):
            g16 = lax.shift_right_logical(i, 4)
            lane = lax.bitwise_and(i, 15)
            tid16 = tid_row[pl.ds(g16 * 16, 16)]
            lane_v = jnp.full((16, 1), lane, dtype=jnp.int32)
            tid_splat = lax.gather(
                tid16, lane_v,
                lax.GatherDimensionNumbers(
                    offset_dims=(), collapsed_slice_dims=(0,),
                    start_index_map=(0,)),
                (1,), mode=lax.GatherScatterMode.PROMISE_IN_BOUNDS)
            is0 = tid_splat == 0
            xs = []
            ssum = jnp.zeros((16,), jnp.float32)
            ssq = jnp.zeros((16,), jnp.float32)
            for r in range(NVR):
                tok = wb[i, pl.ds(r * 16, 16)]
                posr = pb[i, pl.ds(r * 16, 16)]
                ter = jnp.where(is0, const_v[2, pl.ds(r * 16, 16)],
                                const_v[3, pl.ds(r * 16, 16)])
                x = scale * tok + ter + posr
                xs.append(x)
                ssum = ssum + x
                ssq = ssq + x * x
            tsum = jnp.broadcast_to(lax.reduce_sum(ssum, (0,)), (16,))
            tsq = jnp.broadcast_to(lax.reduce_sum(ssq, (0,)), (16,))
            mean = tsum * inv_d
            var = tsq * inv_d - mean * mean
            rstd = _rsqrt_newton(var + jnp.float32(EPS))
            for r in range(NVR):
                gr = const_v[0, pl.ds(r * 16, 16)] * rstd
                wb[i, pl.ds(r * 16, 16)] = (
                    (xs[r] - mean) * gr + const_v[1, pl.ds(r * 16, 16)])
            return carry

        lax.fori_loop(0, CHUNK, token, 0)
        pltpu.sync_copy(wb, out_hbm.at[pl.ds(base_row + j * CHUNK, CHUNK)])
        cp = nxt


def kernel(token_ids, type_ids, word_table, type_table, pos_table,
                 ln_gamma, ln_beta):
    b, s = token_ids.shape
    t_rows = b * s
    n_chunks = t_rows // CHUNK // NW
    idx2d = token_ids.astype(jnp.int32).reshape(t_rows // CHUNK, CHUNK)
    tid2d = type_ids.astype(jnp.int32).reshape(t_rows // CHUNK, CHUNK)
    ttab = type_table * jnp.sqrt(jnp.float32(D))
    fn = pl.kernel(
        _fused_body,
        out_type=jax.ShapeDtypeStruct((t_rows, D), jnp.float32),
        mesh=plsc.VectorSubcoreMesh(core_axis_name="c", subcore_axis_name="s"),
        scratch_types=[
            pltpu.VMEM((n_chunks, CHUNK), jnp.int32),
            pltpu.VMEM((n_chunks, CHUNK), jnp.int32),
            pltpu.VMEM((CHUNK, D), jnp.float32),
            pltpu.VMEM((CHUNK, D), jnp.float32),
            pltpu.VMEM((CHUNK, D), jnp.float32),
            pltpu.VMEM((CHUNK, D), jnp.float32),
            pltpu.VMEM((4, D), jnp.float32),
            pltpu.SemaphoreType.DMA,
            pltpu.SemaphoreType.DMA,
            pltpu.SemaphoreType.DMA,
            pltpu.SemaphoreType.DMA,
        ],
        compiler_params=pltpu.CompilerParams(needs_layout_passes=False),
    )
    out2d = fn(idx2d, tid2d, word_table, ttab, pos_table, ln_gamma, ln_beta)
    return out2d.reshape(b, s, D)


# speedup vs baseline: 2.1556x; 2.1556x over previous
import jax
import jax.numpy as jnp
from jax import lax
from jax.experimental import pallas as pl
from jax.experimental.pallas import tpu as pltpu
from jax.experimental.pallas import tpu_sc as plsc

D = 128
EPS = 1e-12
NC = 2
NS = 16
NW = NC * NS
CHUNK = 128
NVR = D // 16


def _rsqrt_newton(x):
    i = plsc.bitcast(x, jnp.int32)
    i = jnp.int32(0x5F3759DF) - lax.shift_right_arithmetic(i, 1)
    y = plsc.bitcast(i, jnp.float32)
    half = jnp.float32(0.5) * x
    for _ in range(2):
        y = y * (jnp.float32(1.5) - half * y * y)
    return y


def _tree_sum(vs):
    while len(vs) > 1:
        vs = [vs[i] + vs[i + 1] for i in range(0, len(vs) - 1, 2)] + (
            [vs[-1]] if len(vs) % 2 else [])
    return vs[0]


def _fused_body(idx_hbm, tid_hbm, table_hbm, ttab_hbm, pos_hbm, gam_hbm,
                bet_hbm, out_hbm, idx_v, tid_v, wbuf0, wbuf1, pbuf0, pbuf1,
                const_v, wsem0, wsem1, psem0, psem1):
    wid = lax.axis_index("s") * NC + lax.axis_index("c")
    batch, seq_len = idx_hbm.shape
    rows_per_w = seq_len * batch // NW
    runs_per_row = seq_len // rows_per_w
    n_chunks = rows_per_w // CHUNK
    rb = wid // runs_per_row
    pos0 = (wid % runs_per_row) * rows_per_w
    base_row = wid * rows_per_w

    for jj in range(n_chunks):
        pltpu.sync_copy(idx_hbm.at[rb, pl.ds(pos0 + jj * CHUNK, CHUNK)],
                        idx_v.at[jj])
        pltpu.sync_copy(tid_hbm.at[rb, pl.ds(pos0 + jj * CHUNK, CHUNK)],
                        tid_v.at[jj])
    pltpu.sync_copy(gam_hbm, const_v.at[0])
    pltpu.sync_copy(bet_hbm, const_v.at[1])
    pltpu.sync_copy(ttab_hbm.at[0], const_v.at[2])
    pltpu.sync_copy(ttab_hbm.at[1], const_v.at[3])

    scale = jnp.float32(float(D) ** 0.5)
    inv_d = jnp.float32(1.0 / D)
    gdn = lax.GatherDimensionNumbers(
        offset_dims=(), collapsed_slice_dims=(0,), start_index_map=(0,))

    gam_c = [const_v[0, pl.ds(r * 16, 16)] for r in range(NVR)]
    bet_c = [const_v[1, pl.ds(r * 16, 16)] for r in range(NVR)]
    ty0_c = [scale * const_v[2, pl.ds(r * 16, 16)] for r in range(NVR)]
    ty1_c = [scale * const_v[3, pl.ds(r * 16, 16)] for r in range(NVR)]

    wbufs = (wbuf0, wbuf1)
    pbufs = (pbuf0, pbuf1)
    wsems = (wsem0, wsem1)
    psems = (psem0, psem1)

    def start(jj, b):
        pltpu.async_copy(table_hbm.at[idx_v.at[jj]], wbufs[b], wsems[b])
        pltpu.async_copy(pos_hbm.at[pl.ds(pos0 + jj * CHUNK, CHUNK)],
                         pbufs[b], psems[b])

    def wait(jj, b):
        pltpu.make_async_copy(table_hbm.at[idx_v.at[jj]], wbufs[b],
                              wsems[b]).wait()
        pltpu.make_async_copy(pos_hbm.at[pl.ds(pos0 + jj * CHUNK, CHUNK)],
                              pbufs[b], psems[b]).wait()

    def process(jj, b):
        wb = wbufs[b]
        pb = pbufs[b]
        tid_row = tid_v.at[jj]

        @plsc.parallel_loop(0, CHUNK, unroll=2)
        def token(i):
            g16 = lax.shift_right_logical(i, 4)
            lane = lax.bitwise_and(i, 15)
            tid16 = tid_row[pl.ds(g16 * 16, 16)]
            lane_v = jnp.full((16, 1), lane, dtype=jnp.int32)
            tid_splat = lax.gather(
                tid16, lane_v, gdn, (1,),
                mode=lax.GatherScatterMode.PROMISE_IN_BOUNDS)
            is0 = tid_splat == 0
            xs = []
            sqs = []
            for r in range(NVR):
                tok = wb[i, pl.ds(r * 16, 16)]
                posr = pb[i, pl.ds(r * 16, 16)]
                te = jnp.where(is0, ty0_c[r], ty1_c[r])
                x = scale * tok + te + posr
                xs.append(x)
                sqs.append(x * x)
            ssum = _tree_sum(list(xs))
            ssq = _tree_sum(sqs)
            tsum = jnp.broadcast_to(lax.reduce_sum(ssum, (0,)), (16,))
            tsq = jnp.broadcast_to(lax.reduce_sum(ssq, (0,)), (16,))
            mean = tsum * inv_d
            var = tsq * inv_d - mean * mean
            rstd = _rsqrt_newton(var + jnp.float32(EPS))
            for r in range(NVR):
                wb[i, pl.ds(r * 16, 16)] = (
                    (xs[r] - mean) * (gam_c[r] * rstd) + bet_c[r])

        pltpu.sync_copy(wb, out_hbm.at[pl.ds(base_row + jj * CHUNK, CHUNK)])

    start(0, 0)

    def two_chunks(k, carry):
        j0 = 2 * k
        wait(j0, 0)
        start(j0 + 1, 1)
        process(j0, 0)
        wait(j0 + 1, 1)

        @pl.when(j0 + 2 < n_chunks)
        def _():
            start(j0 + 2, 0)

        process(j0 + 1, 1)
        return carry

    lax.fori_loop(0, n_chunks // 2, two_chunks, 0)


def kernel(token_ids, type_ids, word_table, type_table, pos_table,
           ln_gamma, ln_beta):
    b, s = token_ids.shape
    t_rows = b * s
    n_chunks = t_rows // CHUNK // NW
    fn = pl.kernel(
        _fused_body,
        out_type=jax.ShapeDtypeStruct((t_rows, D), jnp.float32),
        mesh=plsc.VectorSubcoreMesh(core_axis_name="c", subcore_axis_name="s"),
        scratch_types=[
            pltpu.VMEM((n_chunks, CHUNK), jnp.int32),
            pltpu.VMEM((n_chunks, CHUNK), jnp.int32),
            pltpu.VMEM((CHUNK, D), jnp.float32),
            pltpu.VMEM((CHUNK, D), jnp.float32),
            pltpu.VMEM((CHUNK, D), jnp.float32),
            pltpu.VMEM((CHUNK, D), jnp.float32),
            pltpu.VMEM((4, D), jnp.float32),
            pltpu.SemaphoreType.DMA,
            pltpu.SemaphoreType.DMA,
            pltpu.SemaphoreType.DMA,
            pltpu.SemaphoreType.DMA,
        ],
        compiler_params=pltpu.CompilerParams(needs_layout_passes=False),
    )
    out2d = fn(token_ids.astype(jnp.int32), type_ids.astype(jnp.int32),
               word_table, type_table, pos_table, ln_gamma, ln_beta)
    return out2d.reshape(b, s, D)
